# baseline (speedup 1.0000x reference)
import jax
import jax.numpy as jnp
import numpy as np
from jax.experimental import pallas as pl

_D = 64
_BLK = 256


def _offs():
    r = np.arange(-1, 2)
    g = np.stack(np.meshgrid(r, r, r, indexing='ij'), axis=-1).reshape(-1, 3)
    return jnp.asarray(g, dtype=jnp.int32)


_OFFSETS = _offs()


def _gather27(x, coords, lookup):
    n = x.shape[0]
    xyz = coords[:, :3]
    cols = []
    for k in range(27):
        nb = xyz + _OFFSETS[k]
        inb = jnp.all((nb >= 0) & (nb < _D), axis=1)
        lin = nb[:, 0] * (_D * _D) + nb[:, 1] * _D + nb[:, 2]
        lin = jnp.where(inb, lin, 0)
        idx = lookup[lin]
        valid = inb & (idx >= 0)
        g = x[jnp.where(valid, idx, 0)]
        g = g * valid[:, None].astype(x.dtype)
        cols.append(g)
    return jnp.concatenate(cols, axis=1)


def _mm_kernel(g_ref, w_ref, b_ref, wm_ref, bm_ref, o_ref):
    t = jnp.dot(g_ref[...], w_ref[...], preferred_element_type=jnp.float32) + b_ref[...]
    o_ref[...] = jnp.dot(t, wm_ref[...], preferred_element_type=jnp.float32) + bm_ref[...]


def kernel(coords, feats, W1, b1, W2, b2, W3, b3, Wm, bm):
    n = coords.shape[0]
    lin = coords[:, 0] * (_D * _D) + coords[:, 1] * _D + coords[:, 2]
    lookup = jnp.full((_D * _D * _D,), -1, dtype=jnp.int32).at[lin].set(
        jnp.arange(n, dtype=jnp.int32))
    g1 = _gather27(feats, coords, lookup)
    x1 = g1 @ W1.reshape(27, 16) + b1
    g2 = _gather27(x1, coords, lookup)
    x2 = g2 @ W2.reshape(27 * 16, 32) + b2
    g3 = _gather27(x2, coords, lookup)

    pad_n = ((n + _BLK - 1) // _BLK) * _BLK
    g3p = jnp.pad(g3, ((0, pad_n - n), (0, 0)))
    out = pl.pallas_call(
        _mm_kernel,
        grid=(pad_n // _BLK,),
        in_specs=[
            pl.BlockSpec((_BLK, 27 * 32), lambda i: (i, 0)),
            pl.BlockSpec((27 * 32, 64), lambda i: (0, 0)),
            pl.BlockSpec((1, 64), lambda i: (0, 0)),
            pl.BlockSpec((64, 3), lambda i: (0, 0)),
            pl.BlockSpec((1, 3), lambda i: (0, 0)),
        ],
        out_specs=pl.BlockSpec((_BLK, 3), lambda i: (i, 0)),
        out_shape=jax.ShapeDtypeStruct((pad_n, 3), jnp.float32),
    )(g3p, W3.reshape(27 * 32, 64), b3.reshape(1, 64), Wm, bm.reshape(1, 3))
    return out[:n]


# speedup vs baseline: 3.9143x; 3.9143x over previous
import functools

import jax
import jax.numpy as jnp
from jax import lax
from jax.experimental import pallas as pl
from jax.experimental.pallas import tpu as pltpu
from jax.experimental.pallas import tpu_sc as plsc

_D = 64
_BLK = 256
_NC, _NS, _L = 2, 16, 16
_NW = _NC * _NS
_SENT = _D * _D * _D
_T = 262656

_OFF = [(dx, dy, dz) for dx in (-1, 0, 1) for dy in (-1, 0, 1) for dz in (-1, 0, 1)]

_SC_PARAMS = pltpu.CompilerParams(use_tc_tiling_on_sc=False)


def _build_lookup(lin_a1, vals, pad_n):
    p1 = pad_n // _NS
    ts = _T // _NS
    mesh = plsc.VectorSubcoreMesh(
        core_axis_name="c", subcore_axis_name="s", num_cores=1)

    @functools.partial(
        pl.kernel,
        out_type=jax.ShapeDtypeStruct((_T,), jnp.int32),
        mesh=mesh,
        compiler_params=_SC_PARAMS,
        scratch_types=[
            pltpu.VMEM((ts,), jnp.int32),
            pltpu.VMEM((p1,), jnp.int32),
            pltpu.VMEM((p1,), jnp.int32),
            pltpu.SemaphoreType.DMA,
        ],
    )
    def k(lin_hbm, vals_hbm, lookup_hbm, fillv, linv, valsv, sem):
        wid = lax.axis_index("s")
        neg1 = jnp.full((_L,), -1, jnp.int32)

        def fill_body(i, c):
            fillv[pl.ds(i * _L, _L)] = neg1
            return c

        lax.fori_loop(0, ts // _L, fill_body, 0)
        pltpu.sync_copy(fillv, lookup_hbm.at[pl.ds(wid * ts, ts)])
        plsc.subcore_barrier()
        pltpu.sync_copy(lin_hbm.at[pl.ds(wid * p1, p1)], linv)
        pltpu.sync_copy(vals_hbm.at[pl.ds(wid * p1, p1)], valsv)
        pltpu.async_copy(valsv, lookup_hbm.at[linv], sem).wait()

    return k(lin_a1, vals)


def _neighbor_idx(lin_a2, lookup, pad_n):
    p = pad_n // _NW
    ng = p // _L
    e = p * 27
    zr = pad_n
    mesh = plsc.VectorSubcoreMesh(core_axis_name="c", subcore_axis_name="s")

    @functools.partial(
        pl.kernel,
        out_type=jax.ShapeDtypeStruct((27 * pad_n,), jnp.int32),
        mesh=mesh,
        compiler_params=_SC_PARAMS,
        scratch_types=[
            pltpu.VMEM((p,), jnp.int32),
            pltpu.VMEM((e,), jnp.int32),
            pltpu.VMEM((e,), jnp.int32),
            pltpu.SemaphoreType.DMA,
        ],
    )
    def k(lin_hbm, lookup_hbm, fidx_hbm, linself, linbuf, rawbuf, sem):
        wid = lax.axis_index("s") * _NC + lax.axis_index("c")
        base = wid * p
        pltpu.sync_copy(lin_hbm.at[pl.ds(base, p)], linself)

        def g_body(g, c):
            lin16 = linself[pl.ds(g * _L, _L)]
            x = jnp.right_shift(lin16, 12)
            y = jnp.bitwise_and(jnp.right_shift(lin16, 6), 63)
            z = jnp.bitwise_and(lin16, 63)
            for kk, (dx, dy, dz) in enumerate(_OFF):
                inb = None
                for comp, dd in ((x, dx), (y, dy), (z, dz)):
                    if dd == -1:
                        m = comp >= 1
                    elif dd == 1:
                        m = comp <= _D - 2
                    else:
                        continue
                    inb = m if inb is None else jnp.logical_and(inb, m)
                nlin = lin16 + (dx * 4096 + dy * 64 + dz)
                if inb is not None:
                    nlin = jnp.where(inb, nlin, _SENT)
                linbuf[pl.ds(kk * p + g * _L, _L)] = nlin
            return c

        lax.fori_loop(0, ng, g_body, 0)
        pltpu.async_copy(lookup_hbm.at[linbuf], rawbuf, sem).wait()

        def f_body(v, c):
            r = rawbuf[pl.ds(v * _L, _L)]
            rawbuf[pl.ds(v * _L, _L)] = jnp.where(r >= 0, r, zr)
            return c

        lax.fori_loop(0, e // _L, f_body, 0)
        for kk in range(27):
            pltpu.sync_copy(rawbuf.at[pl.ds(kk * p, p)],
                            fidx_hbm.at[pl.ds(kk * pad_n + base, p)])

    return k(lin_a2, lookup)


def _gather_rows(x_pad, fidx, pad_n):
    rows_n = fidx.shape[0]
    flat = x_pad.ndim == 1
    c = 1 if flat else x_pad.shape[1]
    p = pad_n // _NW
    out_sds = jax.ShapeDtypeStruct((rows_n,) if flat else (rows_n, c),
                                   jnp.float32)
    mesh = plsc.VectorSubcoreMesh(core_axis_name="c", subcore_axis_name="s")

    @functools.partial(
        pl.kernel,
        out_type=out_sds,
        mesh=mesh,
        compiler_params=_SC_PARAMS,
        scratch_types=[
            pltpu.VMEM((p,), jnp.int32),
            pltpu.VMEM((p,) if flat else (p, c), jnp.float32),
            pltpu.SemaphoreType.DMA,
        ],
    )
    def k(x_hbm, fidx_hbm, g_hbm, idxv, rows, sem):
        wid = lax.axis_index("s") * _NC + lax.axis_index("c")
        base = wid * p
        for kk in range(27):
            off = kk * pad_n + base
            pltpu.sync_copy(fidx_hbm.at[pl.ds(off, p)], idxv)
            pltpu.async_copy(x_hbm.at[idxv], rows, sem).wait()
            pltpu.sync_copy(rows, g_hbm.at[pl.ds(off, p)])

    return k(x_pad, fidx)


def _conv_mm(g3d, w, b, zero_tail, head=None):
    _, pad_n, cin = g3d.shape
    cout = w.shape[2]
    nb = pad_n // _BLK

    if head is None:
        def mk(g_ref, w_ref, b_ref, o_ref):
            pid = pl.program_id(0)

            @pl.when(pid < nb)
            def _():
                acc = jnp.zeros((_BLK, cout), jnp.float32)
                for kk in range(27):
                    acc += jnp.dot(g_ref[kk], w_ref[kk],
                                   preferred_element_type=jnp.float32)
                o_ref[...] = acc + b_ref[...]

            if zero_tail:
                @pl.when(pid == nb)
                def _():
                    o_ref[...] = jnp.zeros_like(o_ref)
    else:
        wm, bm = head

        def mk(g_ref, w_ref, b_ref, wm_ref, bm_ref, o_ref):
            acc = jnp.zeros((_BLK, cout), jnp.float32)
            for kk in range(27):
                acc += jnp.dot(g_ref[kk], w_ref[kk],
                               preferred_element_type=jnp.float32)
            t = acc + b_ref[...]
            o_ref[...] = jnp.dot(t, wm_ref[...],
                                 preferred_element_type=jnp.float32) + bm_ref[...]

    ocols = 3 if head is not None else cout
    in_specs = [
        pl.BlockSpec((27, _BLK, cin), lambda i: (0, jnp.minimum(i, nb - 1), 0)),
        pl.BlockSpec((27, cin, cout), lambda i: (0, 0, 0)),
        pl.BlockSpec((1, cout), lambda i: (0, 0)),
    ]
    args = [g3d, w, b.reshape(1, cout)]
    if head is not None:
        in_specs += [pl.BlockSpec((cout, 3), lambda i: (0, 0)),
                     pl.BlockSpec((1, 3), lambda i: (0, 0))]
        args += [head[0], head[1].reshape(1, 3)]
    grid = (nb + 1,) if zero_tail else (nb,)
    orows = pad_n + _BLK if zero_tail else pad_n
    return pl.pallas_call(
        mk,
        grid=grid,
        in_specs=in_specs,
        out_specs=pl.BlockSpec((_BLK, ocols), lambda i: (i, 0)),
        out_shape=jax.ShapeDtypeStruct((orows, ocols), jnp.float32),
    )(*args)


def kernel(coords, feats, W1, b1, W2, b2, W3, b3, Wm, bm):
    n = coords.shape[0]
    pad_n = -(-n // (_NW * _L)) * (_NW * _L)
    pad = pad_n - n
    lin = coords[:, 0] * (_D * _D) + coords[:, 1] * _D + coords[:, 2]
    lin_a1 = jnp.concatenate(
        [lin, _SENT + 8 + jnp.arange(pad, dtype=jnp.int32)])
    lin_a2 = jnp.concatenate([lin, jnp.full((pad,), _SENT, jnp.int32)])
    vals = jnp.arange(pad_n, dtype=jnp.int32)

    lookup = _build_lookup(lin_a1, vals, pad_n)
    fidx = _neighbor_idx(lin_a2, lookup, pad_n)

    r = pad_n + _BLK
    x1 = jnp.pad(feats.reshape(-1), (0, r - n))
    g1 = _gather_rows(x1, fidx, pad_n)
    x2 = _conv_mm(g1.reshape(27, pad_n, 1), W1, b1, zero_tail=True)
    g2 = _gather_rows(x2, fidx, pad_n)
    x3 = _conv_mm(g2.reshape(27, pad_n, 16), W2, b2, zero_tail=True)
    g3 = _gather_rows(x3, fidx, pad_n)
    out = _conv_mm(g3.reshape(27, pad_n, 32), W3, b3, zero_tail=False,
                   head=(Wm, bm))
    return out[:n]
